# Initial kernel scaffold; baseline (speedup 1.0000x reference)
import jax
import jax.numpy as jnp
from jax.experimental import pallas as pl


def kernel(q, k):
    raise NotImplementedError("write your pallas kernel here")


# speedup vs baseline: 1.6242x; 1.6242x over previous
import jax
import jax.numpy as jnp
from jax.experimental import pallas as pl
from jax.experimental.pallas import tpu as pltpu

_TEMPERATURE = 45.254834
_INV_TEMPERATURE = 1.0 / _TEMPERATURE


def _attn_kernel(q_ref, k_ref, o_ref):
    logits = jax.lax.dot_general(
        q_ref[...],
        k_ref[...],
        (((1,), (1,)), ((), ())),
        preferred_element_type=jnp.float32,
    )
    x = logits * _INV_TEMPERATURE
    m = jnp.max(x, axis=-1, keepdims=True)
    e = jnp.exp(x - m)
    denom = jnp.sum(e, axis=-1, keepdims=True)
    o_ref[...] = e / denom


def kernel(q, k):
    n, d = q.shape
    nk = k.shape[0]
    br = 256
    qb = q.astype(jnp.bfloat16)
    kb = k.astype(jnp.bfloat16)
    return pl.pallas_call(
        _attn_kernel,
        grid=(n // br,),
        in_specs=[
            pl.BlockSpec((br, d), lambda i: (i, 0)),
            pl.BlockSpec((nk, d), lambda i: (0, 0)),
        ],
        out_specs=pl.BlockSpec((br, nk), lambda i: (i, 0)),
        out_shape=jax.ShapeDtypeStruct((n, nk), jnp.float32),
        compiler_params=pltpu.CompilerParams(
            dimension_semantics=("arbitrary",)
        ),
    )(qb, kb)


# speedup vs baseline: 1.6543x; 1.0185x over previous
import jax
import jax.numpy as jnp
from jax.experimental import pallas as pl
from jax.experimental.pallas import tpu as pltpu

_TEMPERATURE = 45.254834
_INV_TEMPERATURE = 1.0 / _TEMPERATURE


def _attn_kernel(q_ref, k_ref, o_ref):
    x = jax.lax.dot_general(
        q_ref[...],
        k_ref[...],
        (((1,), (1,)), ((), ())),
        preferred_element_type=jnp.float32,
    )
    m = jnp.max(x, axis=-1, keepdims=True)
    e = jnp.exp2(x - m)
    r = 1.0 / jnp.sum(e, axis=-1, keepdims=True)
    o_ref[...] = e * r


def kernel(q, k):
    n, d = q.shape
    nk = k.shape[0]
    br = 256
    qb = (q * (1.4426950408889634 * _INV_TEMPERATURE)).astype(jnp.bfloat16)
    kb = k.astype(jnp.bfloat16)
    return pl.pallas_call(
        _attn_kernel,
        grid=(n // br,),
        in_specs=[
            pl.BlockSpec((br, d), lambda i: (i, 0)),
            pl.BlockSpec((nk, d), lambda i: (0, 0)),
        ],
        out_specs=pl.BlockSpec((br, nk), lambda i: (i, 0)),
        out_shape=jax.ShapeDtypeStruct((n, nk), jnp.float32),
        compiler_params=pltpu.CompilerParams(
            dimension_semantics=("arbitrary",)
        ),
    )(qb, kb)
